# Initial kernel scaffold; baseline (speedup 1.0000x reference)
import jax
import jax.numpy as jnp
from jax import lax
from jax.experimental import pallas as pl
from jax.experimental.pallas import tpu as pltpu
from jax.experimental.pallas import tpu_sc as plsc

NC, NS = 2, 16
NW = NC * NS
K = 80
G = 5


def _deg_body(dstr, degp, didx, obuf, zbuf, acc, sem):
    nblk = didx.shape[0]
    n = acc.shape[0]
    rpt = n // NS
    cid = lax.axis_index("c")
    sid = lax.axis_index("s")
    wid = cid * NS + sid
    pltpu.sync_copy(dstr.at[wid], didx)
    ones = jnp.ones((16,), jnp.float32)
    zeros = jnp.zeros((16,), jnp.float32)

    def fill_ones(r, c):
        obuf[r, :] = ones
        return c

    lax.fori_loop(0, obuf.shape[0], fill_ones, 0)

    def fill_zeros(r, c):
        zbuf[r, :] = zeros
        return c

    lax.fori_loop(0, rpt, fill_zeros, 0)
    pltpu.sync_copy(zbuf, acc.at[pl.ds(sid * rpt, rpt)])
    plsc.subcore_barrier()

    def group(gi, c):
        descs = [
            pltpu.async_copy(obuf, acc.at[didx.at[gi * G + j]], sem, add=True)
            for j in range(G)
        ]
        for d in descs:
            d.wait()
        return c

    lax.fori_loop(0, nblk // G, group, 0)
    plsc.subcore_barrier()
    pltpu.sync_copy(acc.at[pl.ds(sid * rpt, rpt)], degp.at[cid, pl.ds(sid * rpt, rpt)])


def _make_deg(n, e):
    nblk = e // (NW * K)
    mesh = plsc.VectorSubcoreMesh(
        core_axis_name="c", subcore_axis_name="s", num_cores=NC, num_subcores=NS
    )
    return pl.kernel(
        _deg_body,
        out_type=jax.ShapeDtypeStruct((NC, n, 16), jnp.float32),
        mesh=mesh,
        scratch_types=[
            pltpu.VMEM((nblk, K), jnp.int32),
            pltpu.VMEM((K, 16), jnp.float32),
            pltpu.VMEM((n // NS, 16), jnp.float32),
            pltpu.VMEM_SHARED((n, 16), jnp.float32),
            pltpu.SemaphoreType.DMA,
        ],
    )


def _scatter_body(g_hbm, srcr, dstr, outp, sidx, didx, msg, zbuf, acc, gsem, ssem):
    nblk = sidx.shape[0]
    n = acc.shape[0]
    d = acc.shape[1]
    rpt = n // NS
    zr = zbuf.shape[0]
    cid = lax.axis_index("c")
    sid = lax.axis_index("s")
    wid = cid * NS + sid
    pltpu.sync_copy(srcr.at[wid], sidx)
    pltpu.sync_copy(dstr.at[wid], didx)
    zeros = jnp.zeros((16,), jnp.float32)

    def fz(r, c):
        for q in range(d // 16):
            zbuf[r, pl.ds(q * 16, 16)] = zeros
        return c

    lax.fori_loop(0, zr, fz, 0)
    for q in range(rpt // zr):
        pltpu.sync_copy(zbuf, acc.at[pl.ds(sid * rpt + q * zr, zr)])
    plsc.subcore_barrier()

    def group(gi, c):
        g0 = gi * G
        gd = [
            pltpu.async_copy(g_hbm.at[sidx.at[g0 + j]], msg.at[j], gsem)
            for j in range(G)
        ]
        sd = []
        for j in range(G):
            gd[j].wait()
            sd.append(
                pltpu.async_copy(msg.at[j], acc.at[didx.at[g0 + j]], ssem, add=True)
            )
        for dsc in sd:
            dsc.wait()
        return c

    lax.fori_loop(0, nblk // G, group, 0)
    plsc.subcore_barrier()
    for q in range(rpt // zr):
        pltpu.sync_copy(
            acc.at[pl.ds(sid * rpt + q * zr, zr)],
            outp.at[cid, pl.ds(sid * rpt + q * zr, zr)],
        )


def _make_scatter(n, d, e):
    nblk = e // (NW * K)
    zr = 125
    mesh = plsc.VectorSubcoreMesh(
        core_axis_name="c", subcore_axis_name="s", num_cores=NC, num_subcores=NS
    )
    return pl.kernel(
        _scatter_body,
        out_type=jax.ShapeDtypeStruct((NC, n, d), jnp.float32),
        mesh=mesh,
        scratch_types=[
            pltpu.VMEM((nblk, K), jnp.int32),
            pltpu.VMEM((nblk, K), jnp.int32),
            pltpu.VMEM((G, K, d), jnp.float32),
            pltpu.VMEM((zr, d), jnp.float32),
            pltpu.VMEM_SHARED((n, d), jnp.float32),
            pltpu.SemaphoreType.DMA,
            pltpu.SemaphoreType.DMA,
        ],
    )


def _m1_body(x_ref, w1t_ref, degp_ref, g1_ref, dinv_ref):
    deg = degp_ref[0, :, 0:1] + degp_ref[1, :, 0:1] + 1.0
    dinv = lax.rsqrt(deg)
    h = jnp.dot(x_ref[...], w1t_ref[...], preferred_element_type=jnp.float32)
    g1_ref[...] = h * dinv
    dinv_ref[...] = jnp.broadcast_to(dinv, dinv_ref.shape)


def _m2_body(s1p_ref, g1_ref, dinv_ref, b1_ref, w2t_ref, x1_ref, g2_ref):
    dinv = dinv_ref[:, 0:1]
    tot = (s1p_ref[0] + s1p_ref[1] + g1_ref[...]) * dinv + b1_ref[...]
    x1 = jnp.maximum(tot, 0.0)
    x1_ref[...] = x1
    g2_ref[...] = jnp.dot(x1, w2t_ref[...], preferred_element_type=jnp.float32) * dinv


def _m3_body(s2p_ref, g2_ref, dinv_ref, b2_ref, x2_ref):
    dinv = dinv_ref[:, 0:1]
    x2_ref[...] = (s2p_ref[0] + s2p_ref[1] + g2_ref[...]) * dinv + b2_ref[...]


def kernel(x, edge_index, W1, b1, W2, b2):
    n, din = x.shape
    dm = W1.shape[0]
    e = edge_index.shape[1]
    assert e % (NW * K * G) == 0 and n % (NS * 125) == 0
    nblk = e // (NW * K)
    src = edge_index[0].reshape(NW, nblk, K)
    dst = edge_index[1].reshape(NW, nblk, K)
    w1t = W1.T
    w2t = W2.T
    b1r = b1.reshape(1, dm)
    b2r = b2.reshape(1, dm)

    degp = _make_deg(n, e)(dst)

    B = 2000
    grid = (n // B,)
    g1, dinv16 = pl.pallas_call(
        _m1_body,
        grid=grid,
        in_specs=[
            pl.BlockSpec((B, din), lambda i: (i, 0)),
            pl.BlockSpec((din, dm), lambda i: (0, 0)),
            pl.BlockSpec((NC, B, 16), lambda i: (0, i, 0)),
        ],
        out_specs=[
            pl.BlockSpec((B, dm), lambda i: (i, 0)),
            pl.BlockSpec((B, 16), lambda i: (i, 0)),
        ],
        out_shape=[
            jax.ShapeDtypeStruct((n, dm), jnp.float32),
            jax.ShapeDtypeStruct((n, 16), jnp.float32),
        ],
    )(x, w1t, degp)

    scat = _make_scatter(n, dm, e)
    s1p = scat(g1, src, dst)

    x1, g2 = pl.pallas_call(
        _m2_body,
        grid=grid,
        in_specs=[
            pl.BlockSpec((NC, B, dm), lambda i: (0, i, 0)),
            pl.BlockSpec((B, dm), lambda i: (i, 0)),
            pl.BlockSpec((B, 16), lambda i: (i, 0)),
            pl.BlockSpec((1, dm), lambda i: (0, 0)),
            pl.BlockSpec((dm, dm), lambda i: (0, 0)),
        ],
        out_specs=[
            pl.BlockSpec((B, dm), lambda i: (i, 0)),
            pl.BlockSpec((B, dm), lambda i: (i, 0)),
        ],
        out_shape=[
            jax.ShapeDtypeStruct((n, dm), jnp.float32),
            jax.ShapeDtypeStruct((n, dm), jnp.float32),
        ],
    )(s1p, g1, dinv16, b1r, w2t)

    s2p = scat(g2, src, dst)

    x2 = pl.pallas_call(
        _m3_body,
        grid=grid,
        in_specs=[
            pl.BlockSpec((NC, B, dm), lambda i: (0, i, 0)),
            pl.BlockSpec((B, dm), lambda i: (i, 0)),
            pl.BlockSpec((B, 16), lambda i: (i, 0)),
            pl.BlockSpec((1, dm), lambda i: (0, 0)),
        ],
        out_specs=pl.BlockSpec((B, dm), lambda i: (i, 0)),
        out_shape=jax.ShapeDtypeStruct((n, dm), jnp.float32),
    )(s2p, g2, dinv16, b2r)

    return (x2, x1)


# speedup vs baseline: 24.7830x; 24.7830x over previous
import jax
import jax.numpy as jnp
from jax import lax
from jax.experimental import pallas as pl
from jax.experimental.pallas import tpu as pltpu
from jax.experimental.pallas import tpu_sc as plsc

NC, NS = 2, 16
NW = NC * NS
K = 40
G = 5


def _row_split(n):
    base = (n // (NS * 8)) * 8
    last = n - (NS - 1) * base
    return base, last


def _deg_body(dstr, zer, obuf_h, degp, didxb, obuf, acc, isem, sem):
    ngrp = dstr.shape[1]
    n = acc.shape[0]
    base, last = _row_split(n)
    cid = lax.axis_index("c")
    sid = lax.axis_index("s")
    wid = cid * NS + sid
    r0 = sid * base
    pltpu.async_copy(dstr.at[wid, 0], didxb.at[0], isem)
    pltpu.sync_copy(obuf_h, obuf)

    @pl.when(sid < NS - 1)
    def _():
        pltpu.sync_copy(zer.at[pl.ds(r0, base)], acc.at[pl.ds(r0, base)])

    @pl.when(sid == NS - 1)
    def _():
        pltpu.sync_copy(zer.at[pl.ds(r0, last)], acc.at[pl.ds(r0, last)])

    plsc.subcore_barrier()

    def pair(pi, c):
        for slot in range(2):
            gi = pi * 2 + slot
            pltpu.make_async_copy(dstr.at[wid, gi], didxb.at[slot], isem).wait()

            @pl.when(gi + 1 < ngrp)
            def _():
                pltpu.async_copy(dstr.at[wid, gi + 1], didxb.at[1 - slot], isem)

            for j in range(G):
                pltpu.sync_copy(obuf, acc.at[didxb.at[slot, j]], add=True)
        return c

    lax.fori_loop(0, ngrp // 2, pair, 0)
    plsc.subcore_barrier()

    @pl.when(sid < NS - 1)
    def _():
        pltpu.sync_copy(acc.at[pl.ds(r0, base)], degp.at[cid, pl.ds(r0, base)])

    @pl.when(sid == NS - 1)
    def _():
        pltpu.sync_copy(acc.at[pl.ds(r0, last)], degp.at[cid, pl.ds(r0, last)])


def _make_deg(n, d):
    mesh = plsc.VectorSubcoreMesh(
        core_axis_name="c", subcore_axis_name="s", num_cores=NC, num_subcores=NS
    )
    return pl.kernel(
        _deg_body,
        out_type=jax.ShapeDtypeStruct((NC, n, d), jnp.float32),
        mesh=mesh,
        scratch_types=[
            pltpu.VMEM((2, G, K), jnp.int32),
            pltpu.VMEM((K, d), jnp.float32),
            pltpu.VMEM_SHARED((n, d), jnp.float32),
            pltpu.SemaphoreType.DMA,
            pltpu.SemaphoreType.DMA,
        ],
    )


def _scatter_body(g_hbm, srcr, dstr, zer, outp, sidxb, didxb, msg, acc, isem, gsem, ssem):
    ngrp = srcr.shape[1]
    n = acc.shape[0]
    base, last = _row_split(n)
    cid = lax.axis_index("c")
    sid = lax.axis_index("s")
    wid = cid * NS + sid
    r0 = sid * base
    pltpu.async_copy(srcr.at[wid, 0], sidxb.at[0], isem)
    pltpu.async_copy(dstr.at[wid, 0], didxb.at[0], isem)

    @pl.when(sid < NS - 1)
    def _():
        pltpu.sync_copy(zer.at[pl.ds(r0, base)], acc.at[pl.ds(r0, base)])

    @pl.when(sid == NS - 1)
    def _():
        pltpu.sync_copy(zer.at[pl.ds(r0, last)], acc.at[pl.ds(r0, last)])

    plsc.subcore_barrier()

    def pair(pi, c):
        for slot in range(2):
            gi = pi * 2 + slot
            pltpu.make_async_copy(srcr.at[wid, gi], sidxb.at[slot], isem).wait()
            pltpu.make_async_copy(dstr.at[wid, gi], didxb.at[slot], isem).wait()

            @pl.when(gi + 1 < ngrp)
            def _():
                pltpu.async_copy(srcr.at[wid, gi + 1], sidxb.at[1 - slot], isem)
                pltpu.async_copy(dstr.at[wid, gi + 1], didxb.at[1 - slot], isem)

            gd = [
                pltpu.async_copy(g_hbm.at[sidxb.at[slot, j]], msg.at[j], gsem)
                for j in range(G)
            ]
            sd = []
            for j in range(G):
                gd[j].wait()
                sd.append(
                    pltpu.async_copy(msg.at[j], acc.at[didxb.at[slot, j]], ssem, add=True)
                )
            for dsc in sd:
                dsc.wait()
        return c

    lax.fori_loop(0, ngrp // 2, pair, 0)
    plsc.subcore_barrier()

    @pl.when(sid < NS - 1)
    def _():
        pltpu.sync_copy(acc.at[pl.ds(r0, base)], outp.at[cid, pl.ds(r0, base)])

    @pl.when(sid == NS - 1)
    def _():
        pltpu.sync_copy(acc.at[pl.ds(r0, last)], outp.at[cid, pl.ds(r0, last)])


def _make_scatter(n, d):
    mesh = plsc.VectorSubcoreMesh(
        core_axis_name="c", subcore_axis_name="s", num_cores=NC, num_subcores=NS
    )
    return pl.kernel(
        _scatter_body,
        out_type=jax.ShapeDtypeStruct((NC, n, d), jnp.float32),
        mesh=mesh,
        scratch_types=[
            pltpu.VMEM((2, G, K), jnp.int32),
            pltpu.VMEM((2, G, K), jnp.int32),
            pltpu.VMEM((G, K, d), jnp.float32),
            pltpu.VMEM_SHARED((n, d), jnp.float32),
            pltpu.SemaphoreType.DMA,
            pltpu.SemaphoreType.DMA,
            pltpu.SemaphoreType.DMA,
        ],
    )


def _m1_body(x_ref, w1t_ref, degp_ref, g1_ref, dinv_ref):
    deg = degp_ref[0, :, 0:1] + degp_ref[1, :, 0:1] + 1.0
    dinv = lax.rsqrt(deg)
    h = jnp.dot(x_ref[...], w1t_ref[...], preferred_element_type=jnp.float32)
    g1_ref[...] = h * dinv
    dinv_ref[...] = jnp.broadcast_to(dinv, dinv_ref.shape)


def _m2_body(s1p_ref, g1_ref, dinv_ref, b1_ref, w2t_ref, x1_ref, g2_ref):
    dinv = dinv_ref[:, 0:1]
    tot = (s1p_ref[0] + s1p_ref[1] + g1_ref[...]) * dinv + b1_ref[...]
    x1 = jnp.maximum(tot, 0.0)
    x1_ref[...] = x1
    g2_ref[...] = jnp.dot(x1, w2t_ref[...], preferred_element_type=jnp.float32) * dinv


def _m3_body(s2p_ref, g2_ref, dinv_ref, b2_ref, x2_ref):
    dinv = dinv_ref[:, 0:1]
    x2_ref[...] = (s2p_ref[0] + s2p_ref[1] + g2_ref[...]) * dinv + b2_ref[...]


def kernel(x, edge_index, W1, b1, W2, b2):
    n, din = x.shape
    dm = W1.shape[0]
    e = edge_index.shape[1]
    assert e % (NW * K * G * 2) == 0 and n % 8 == 0
    nblk = e // (NW * K)
    src4 = edge_index[0].reshape(NW, nblk // G, G, K)
    dst4 = edge_index[1].reshape(NW, nblk // G, G, K)
    w1t = W1.T
    w2t = W2.T
    b1r = b1.reshape(1, dm)
    b2r = b2.reshape(1, dm)
    zerd = jnp.zeros((n, dm), jnp.float32)
    ones = jnp.ones((K, dm), jnp.float32)

    degp = _make_deg(n, dm)(dst4, zerd, ones)

    B = 2000
    grid = (n // B,)
    g1, dinv16 = pl.pallas_call(
        _m1_body,
        grid=grid,
        in_specs=[
            pl.BlockSpec((B, din), lambda i: (i, 0)),
            pl.BlockSpec((din, dm), lambda i: (0, 0)),
            pl.BlockSpec((NC, B, dm), lambda i: (0, i, 0)),
        ],
        out_specs=[
            pl.BlockSpec((B, dm), lambda i: (i, 0)),
            pl.BlockSpec((B, 16), lambda i: (i, 0)),
        ],
        out_shape=[
            jax.ShapeDtypeStruct((n, dm), jnp.float32),
            jax.ShapeDtypeStruct((n, 16), jnp.float32),
        ],
    )(x, w1t, degp)

    scat = _make_scatter(n, dm)
    s1p = scat(g1, src4, dst4, zerd)

    x1, g2 = pl.pallas_call(
        _m2_body,
        grid=grid,
        in_specs=[
            pl.BlockSpec((NC, B, dm), lambda i: (0, i, 0)),
            pl.BlockSpec((B, dm), lambda i: (i, 0)),
            pl.BlockSpec((B, 16), lambda i: (i, 0)),
            pl.BlockSpec((1, dm), lambda i: (0, 0)),
            pl.BlockSpec((dm, dm), lambda i: (0, 0)),
        ],
        out_specs=[
            pl.BlockSpec((B, dm), lambda i: (i, 0)),
            pl.BlockSpec((B, dm), lambda i: (i, 0)),
        ],
        out_shape=[
            jax.ShapeDtypeStruct((n, dm), jnp.float32),
            jax.ShapeDtypeStruct((n, dm), jnp.float32),
        ],
    )(s1p, g1, dinv16, b1r, w2t)

    s2p = scat(g2, src4, dst4, zerd)

    x2 = pl.pallas_call(
        _m3_body,
        grid=grid,
        in_specs=[
            pl.BlockSpec((NC, B, dm), lambda i: (0, i, 0)),
            pl.BlockSpec((B, dm), lambda i: (i, 0)),
            pl.BlockSpec((B, 16), lambda i: (i, 0)),
            pl.BlockSpec((1, dm), lambda i: (0, 0)),
        ],
        out_specs=pl.BlockSpec((B, dm), lambda i: (i, 0)),
        out_shape=jax.ShapeDtypeStruct((n, dm), jnp.float32),
    )(s2p, g2, dinv16, b2r)

    return (x2, x1)
